# baseline (speedup 1.0000x reference)
import functools

import jax
import jax.numpy as jnp
from jax import lax
from jax.experimental import pallas as pl
from jax.experimental.pallas import tpu as pltpu
from jax.experimental.pallas import tpu_sc as plsc

VOCAB = 1000000
EMBED = 64
BATCH = 16384
HIST = 50

B = BATCH * HIST
NC, NS = 2, 16
NW = NC * NS
CHUNK = 256
CPH = BATCH // CHUNK
NCHUNK = B // CHUNK
G_PER_W = NCHUNK // NW
NBUF = 2
LG = CHUNK // 16


def _make_gather():
    mesh = plsc.VectorSubcoreMesh(core_axis_name="c", subcore_axis_name="s")

    @functools.partial(
        pl.kernel,
        mesh=mesh,
        out_type=jax.ShapeDtypeStruct((HIST, 8, BATCH // 128, 8, 128),
                                      jnp.float32),
        compiler_params=pltpu.CompilerParams(use_tc_tiling_on_sc=False,
                                             needs_layout_passes=False,
                                             disable_bounds_checks=True),
        scratch_types=(
            [pltpu.VMEM((CHUNK,), jnp.int32) for _ in range(NBUF)]
            + [pltpu.VMEM((CHUNK, EMBED), jnp.float32) for _ in range(NBUF)]
            + [pltpu.VMEM((8, CHUNK // 128, 8, 128), jnp.float32)
               for _ in range(NBUF)]
            + [pltpu.SemaphoreType.DMA for _ in range(NBUF)]
            + [pltpu.SemaphoreType.DMA for _ in range(NBUF)]
        ),
    )
    def gather_kernel(idx_hbm, table_hbm, out_hbm,
                      idx0, idx1, rows0, rows1, colt0, colt1,
                      gsem0, gsem1, wsem0, wsem1):
        wid = lax.axis_index("s") * NC + lax.axis_index("c")
        idx_v = [idx0, idx1]
        rows_v = [rows0, rows1]
        colt_v = [colt0, colt1]
        gsems = [gsem0, gsem1]
        wsems = [wsem0, wsem1]
        iota16 = lax.iota(jnp.int32, 16)

        def start(i, s):
            c = wid + i * NW
            off = c * CHUNK
            pltpu.sync_copy(idx_hbm.at[pl.ds(off, CHUNK)], idx_v[s])
            pltpu.async_copy(table_hbm.at[idx_v[s]], rows_v[s], gsems[s])

        def drain(i, s, first):
            c = wid + i * NW
            h = c // CPH
            b0 = (c % CPH) * CHUNK
            pltpu.make_async_copy(table_hbm.at[idx_v[s]], rows_v[s],
                                  gsems[s]).wait()
            if not first:
                pltpu.make_async_copy(colt_v[s], out_hbm.at[0, :,
                                      pl.ds(0, CHUNK // 128)], wsems[s]).wait()

            def jbody(j, carry):
                rj = iota16 + j * 16
                tb = j // 8
                b16 = 16 * (j - tb * 8)
                for e in range(EMBED):
                    ce = jnp.zeros((16,), jnp.int32) + e
                    v = plsc.load_gather(rows_v[s], [rj, ce])
                    colt_v[s][e // 8, tb, e % 8, pl.ds(b16, 16)] = v
                return carry

            lax.fori_loop(0, LG, jbody, 0)
            pltpu.async_copy(colt_v[s],
                             out_hbm.at[h, :, pl.ds(b0 // 128, CHUNK // 128)],
                             wsems[s])

        start(0, 0)
        start(1, 1)
        drain(0, 0, True)
        start(2, 0)
        drain(1, 1, True)

        def body(i, carry):
            for s in range(NBUF):
                @pl.when((i % NBUF) == s)
                def _():
                    start(i, s)
            for s in range(NBUF):
                @pl.when(((i - 1) % NBUF) == s)
                def _():
                    drain(i - 1, s, False)
            return carry

        lax.fori_loop(3, G_PER_W, body, 0)
        for s in range(NBUF):
            @pl.when(((G_PER_W - 1) % NBUF) == s)
            def _():
                drain(G_PER_W - 1, s, False)
        for s in range(NBUF):
            pltpu.make_async_copy(colt_v[s], out_hbm.at[0, :,
                                  pl.ds(0, CHUNK // 128)], wsems[s]).wait()

    return gather_kernel


_gather = _make_gather()


def kernel(data, iword_indicator, iword_numerals, ivectors_weight):
    idx = data.T.reshape(-1).astype(jnp.int32)
    out5 = _gather(idx, ivectors_weight)
    embed = out5.transpose(2, 4, 0, 1, 3).reshape(BATCH, HIST, EMBED)
    if iword_numerals.shape[0] == 0:
        return embed
    numerals = jnp.sign(iword_numerals) * jnp.log(jnp.abs(iword_numerals) + 1.0)
    ne = jnp.ones((EMBED, numerals.shape[0]), jnp.float32).at[0].set(numerals)
    ne = ne.T / (EMBED * 2)
    flat2 = embed.reshape(-1, EMBED)
    mask = iword_indicator.reshape(-1)
    pos = jnp.nonzero(mask, size=iword_numerals.shape[0])[0]
    return flat2.at[pos].set(ne).reshape(embed.shape)


# speedup vs baseline: 1.5449x; 1.5449x over previous
import functools

import jax
import jax.numpy as jnp
from jax import lax
from jax.experimental import pallas as pl
from jax.experimental.pallas import tpu as pltpu
from jax.experimental.pallas import tpu_sc as plsc

VOCAB = 1000000
EMBED = 64
BATCH = 16384
HIST = 50

B = BATCH * HIST
NC, NS = 2, 16
NW = NC * NS
CHUNK = 512
CPH = BATCH // CHUNK
NCHUNK = B // CHUNK
PER_W = NCHUNK // NW
NBUF = 2


def _make_gather():
    mesh = plsc.VectorSubcoreMesh(core_axis_name="c", subcore_axis_name="s")

    @functools.partial(
        pl.kernel,
        mesh=mesh,
        out_type=jax.ShapeDtypeStruct((HIST, BATCH, EMBED), jnp.float32),
        compiler_params=pltpu.CompilerParams(use_tc_tiling_on_sc=False),
        scratch_types=(
            [pltpu.VMEM((CHUNK,), jnp.int32) for _ in range(NBUF)]
            + [pltpu.VMEM((CHUNK, EMBED), jnp.float32) for _ in range(NBUF)]
            + [pltpu.SemaphoreType.DMA for _ in range(NBUF)]
        ),
    )
    def gather_kernel(idx_hbm, table_hbm, out_hbm,
                      idx0, idx1, rows0, rows1, gsem0, gsem1):
        wid = lax.axis_index("s") * NC + lax.axis_index("c")
        idx_v = [idx0, idx1]
        rows_v = [rows0, rows1]
        gsems = [gsem0, gsem1]

        def start(i, s):
            c = wid + i * NW
            off = c * CHUNK
            pltpu.sync_copy(idx_hbm.at[pl.ds(off, CHUNK)], idx_v[s])
            pltpu.async_copy(table_hbm.at[idx_v[s]], rows_v[s], gsems[s])

        def drain(i, s):
            c = wid + i * NW
            h = c // CPH
            b0 = (c % CPH) * CHUNK
            pltpu.make_async_copy(table_hbm.at[idx_v[s]], rows_v[s],
                                  gsems[s]).wait()
            pltpu.sync_copy(rows_v[s], out_hbm.at[h, pl.ds(b0, CHUNK)])

        start(0, 0)

        def body(i, carry):
            for s in range(NBUF):
                @pl.when((i % NBUF) == s)
                def _():
                    start(i, s)
            for s in range(NBUF):
                @pl.when(((i - 1) % NBUF) == s)
                def _():
                    drain(i - 1, s)
            return carry

        lax.fori_loop(1, PER_W, body, 0)
        for s in range(NBUF):
            @pl.when(((PER_W - 1) % NBUF) == s)
            def _():
                drain(PER_W - 1, s)

    return gather_kernel


_gather = _make_gather()


def kernel(data, iword_indicator, iword_numerals, ivectors_weight):
    idx = data.T.reshape(-1).astype(jnp.int32)
    out3 = _gather(idx, ivectors_weight)
    embed = out3.transpose(1, 0, 2)
    if iword_numerals.shape[0] == 0:
        return embed
    numerals = jnp.sign(iword_numerals) * jnp.log(jnp.abs(iword_numerals) + 1.0)
    ne = jnp.ones((EMBED, numerals.shape[0]), jnp.float32).at[0].set(numerals)
    ne = ne.T / (EMBED * 2)
    flat2 = embed.reshape(-1, EMBED)
    mask = iword_indicator.reshape(-1)
    pos = jnp.nonzero(mask, size=iword_numerals.shape[0])[0]
    return flat2.at[pos].set(ne).reshape(embed.shape)
